# baseline (speedup 1.0000x reference)
import jax
import jax.numpy as jnp
from jax import lax
from jax.experimental import pallas as pl
from jax.experimental.pallas import tpu as pltpu
from jax.experimental.pallas import tpu_sc as plsc

NC = 2
NS = 16
L = 16
NF = 125000
NFH = 62504
OFF1 = NF - NFH
B = 16384
BH = B // NC
CH = 512
NCH = BH // CH


def _sc_body(rigT_hbm, camT_hbm, projT_hbm, idx_hbm, cw_out, proj_out,
             plane_v, buf_v, gath_v, vin_v, idxq_v, outc_v, outp_v,
             camT_v, projT_v, shared_v, sem):
    h = lax.axis_index("c")
    t = lax.axis_index("s")

    pltpu.sync_copy(camT_hbm, camT_v)
    pltpu.sync_copy(projT_hbm, projT_v)

    off_v = jnp.full((L,), OFF1, dtype=jnp.int32)
    for half in range(2):
        pltpu.sync_copy(rigT_hbm.at[t, pl.ds(half * OFF1, NFH)], plane_v)

        def chunk_g(q, _, half=half):
            pltpu.sync_copy(idx_hbm.at[h, q], buf_v)
            for g in range(CH // L):
                v = buf_v[pl.ds(g * L, L)]
                fid = lax.shift_right_logical(v, 3)
                if half == 0:
                    loc = jnp.minimum(fid, NFH - 1)
                    gath_v[pl.ds(q * CH + g * L, L)] = plsc.load_gather(
                        plane_v, [loc])
                else:
                    m = fid >= jnp.full((L,), NFH, dtype=jnp.int32)
                    loc = jnp.maximum(fid - off_v, 0)
                    got = plsc.load_gather(plane_v, [loc])
                    prev = gath_v[pl.ds(q * CH + g * L, L)]
                    gath_v[pl.ds(q * CH + g * L, L)] = jnp.where(m, got, prev)
            return 0

        lax.fori_loop(0, NCH, chunk_g, 0)

    pltpu.sync_copy(gath_v, shared_v.at[t])
    plsc.subcore_barrier()

    i_s = lax.shift_right_logical(t, 2)
    k_s = lax.bitwise_and(t, 3)

    def chunk_cw(q, _):
        pltpu.sync_copy(idx_hbm.at[h, q], idxq_v)
        for j in range(4):
            pltpu.sync_copy(shared_v.at[4 * j + k_s, pl.ds(q * CH, CH)],
                            vin_v.at[j])
        for g in range(CH // L):
            v = idxq_v[pl.ds(g * L, L)]
            cid8 = lax.bitwise_and(v, 7)
            base = jnp.full((L,), 8 * (4 * i_s), dtype=jnp.int32) + cid8
            acc = None
            for j in range(4):
                aj = plsc.load_gather(camT_v, [base + 8 * j])
                term = aj * vin_v[j, pl.ds(g * L, L)]
                acc = term if acc is None else acc + term
            outc_v[g // 8, pl.ds((g % 8) * L, L)] = acc
        pltpu.sync_copy(
            outc_v, cw_out.at[i_s, pl.ds(h * (BH // 128) + q * 4, 4), k_s])
        return 0

    lax.fori_loop(0, NCH, chunk_cw, 0)

    pi_s = lax.div(t, 3)
    pj_s = t - pi_s * 3

    @pl.when(t < 9)
    def _():
        def chunk_pj(q, _):
            pltpu.sync_copy(idx_hbm.at[h, q], idxq_v)
            for g in range(CH // L):
                v = idxq_v[pl.ds(g * L, L)]
                cid8 = lax.bitwise_and(v, 7)
                pbase = jnp.full((L,), 8 * (3 * pi_s + pj_s), jnp.int32)
                outp_v[g // 8, pl.ds((g % 8) * L, L)] = plsc.load_gather(
                    projT_v, [pbase + cid8])
            pltpu.sync_copy(
                outp_v,
                proj_out.at[pi_s, pl.ds(h * (BH // 128) + q * 4, 4), pj_s])
            return 0

        lax.fori_loop(0, NCH, chunk_pj, 0)


def _sc_call(rigT, camT, projT, idx3):
    mesh = plsc.VectorSubcoreMesh(core_axis_name="c", subcore_axis_name="s")
    f = pl.kernel(
        _sc_body,
        out_type=[
            jax.ShapeDtypeStruct((4, B // 128, 4, 128), jnp.float32),
            jax.ShapeDtypeStruct((3, B // 128, 4, 128), jnp.float32),
        ],
        mesh=mesh,
        scratch_types=[
            pltpu.VMEM((NFH,), jnp.float32),
            pltpu.VMEM((CH,), jnp.int32),
            pltpu.VMEM((BH,), jnp.float32),
            pltpu.VMEM((4, CH), jnp.float32),
            pltpu.VMEM((CH,), jnp.int32),
            pltpu.VMEM((4, 128), jnp.float32),
            pltpu.VMEM((4, 128), jnp.float32),
            pltpu.VMEM((128,), jnp.float32),
            pltpu.VMEM((128,), jnp.float32),
            pltpu.VMEM_SHARED((NS, BH), jnp.float32),
            pltpu.SemaphoreType.DMA,
        ],
        compiler_params=pltpu.CompilerParams(
            use_tc_tiling_on_sc=False, needs_layout_passes=False),
    )
    return f(rigT, camT, projT, idx3)


def kernel(rig_t_world, camera_t_rig, projection, image_idx):
    nf = rig_t_world.shape[0]
    ncam = camera_t_rig.shape[0]
    b = image_idx.shape[0]

    rigT = jnp.transpose(rig_t_world, (1, 2, 0)).reshape(16, nf)
    camT = jnp.concatenate(
        [jnp.transpose(camera_t_rig, (1, 2, 0)).reshape(16 * ncam),
         jnp.zeros((128 - 16 * ncam,), jnp.float32)])
    projT = jnp.concatenate(
        [jnp.transpose(projection, (1, 2, 0)).reshape(9 * ncam),
         jnp.zeros((128 - 9 * ncam,), jnp.float32)])
    idx3 = image_idx.reshape(NC, NCH, CH)

    cwp, prjp = _sc_call(rigT, camT, projT, idx3)

    camera_t_world = cwp.transpose(1, 3, 0, 2).reshape(b, 4, 4)
    proj = prjp[:, :, :3, :].transpose(1, 3, 0, 2).reshape(b, 3, 3)
    return (camera_t_world, proj)


# speedup vs baseline: 1.0580x; 1.0580x over previous
import jax
import jax.numpy as jnp
from jax import lax
from jax.experimental import pallas as pl
from jax.experimental.pallas import tpu as pltpu
from jax.experimental.pallas import tpu_sc as plsc

NC = 2
NS = 16
L = 16
NW = NC * NS
CHUNK = 128
NCHUNK = 4
BPW = CHUNK * NCHUNK


def _take16(vec, idx):
    return lax.gather(
        vec, idx[:, None],
        lax.GatherDimensionNumbers(
            offset_dims=(), collapsed_slice_dims=(0,),
            start_index_map=(0,)),
        (1,), mode=lax.GatherScatterMode.PROMISE_IN_BOUNDS)


def _sc_body(rig_hbm, cam_hbm, proj_hbm, idx_hbm, cw_out, proj_out,
             idx_v, fid_v, cid_v, rrows_v, camtab_v, projtab_v,
             cwT_v, prjT_v, sem):
    wid = lax.axis_index("s") * NC + lax.axis_index("c")

    pltpu.sync_copy(idx_hbm.at[wid], idx_v)

    for c in range(NCHUNK):
        for i in range(CHUNK // L):
            v = idx_v[c, pl.ds(i * L, L)]
            fid_v[c, pl.ds(i * L, L)] = lax.shift_right_logical(v, 3)
            cid_v[c, pl.ds(i * L, L)] = lax.bitwise_and(v, 7)

    handles = [
        pltpu.async_copy(rig_hbm.at[fid_v.at[c]], rrows_v.at[c], sem)
        for c in range(NCHUNK)
    ]
    pltpu.sync_copy(cam_hbm, camtab_v)
    pltpu.sync_copy(proj_hbm, projtab_v)

    iota = lax.iota(jnp.int32, L)
    idx_a = [lax.bitwise_and(iota, 12) + j for j in range(4)]
    idx_r = [lax.bitwise_and(iota, 3) + 4 * j for j in range(4)]
    evecs = [jnp.full((L,), e, dtype=jnp.int32) for e in range(L)]
    cw_i = lax.shift_right_logical(iota, 2)
    cw_j = lax.bitwise_and(iota, 3)
    pr_i = lax.div(iota, jnp.full((L,), 3, jnp.int32))
    pr_j = iota - pr_i * 3
    pr_msk = iota < 9

    for c in range(NCHUNK):
        handles[c].wait()

        def group(g, _, c=c):
            cidv = cid_v[c, pl.ds(g * L, L)]
            for e in range(L):
                b = g * L + e
                bvec = jnp.full((L,), b, dtype=jnp.int32)
                cb16 = lax.shift_left(_take16(cidv, evecs[e]), 4)
                rrow = rrows_v[c, b]
                prow = plsc.load_gather(projtab_v, [cb16 + iota])
                plsc.store_scatter(prjT_v.at[c], [pr_i, pr_j, bvec], prow,
                                   mask=pr_msk)
                acc = None
                for j in range(4):
                    aj = plsc.load_gather(camtab_v, [cb16 + idx_a[j]])
                    term = aj * _take16(rrow, idx_r[j])
                    acc = term if acc is None else acc + term
                plsc.store_scatter(cwT_v.at[c], [cw_i, cw_j, bvec], acc)
            return 0

        lax.fori_loop(0, CHUNK // L, group, 0)

        bb = wid * NCHUNK + c
        pltpu.sync_copy(cwT_v.at[c], cw_out.at[:, bb])
        pltpu.sync_copy(prjT_v.at[c], proj_out.at[:, bb])


def _sc_call(rig, cam, proj, idx4):
    mesh = plsc.VectorSubcoreMesh(core_axis_name="c", subcore_axis_name="s")
    f = pl.kernel(
        _sc_body,
        out_type=[
            jax.ShapeDtypeStruct((4, NW * NCHUNK, 4, CHUNK), jnp.float32),
            jax.ShapeDtypeStruct((3, NW * NCHUNK, 4, CHUNK), jnp.float32),
        ],
        mesh=mesh,
        scratch_types=[
            pltpu.VMEM((NCHUNK, CHUNK), jnp.int32),
            pltpu.VMEM((NCHUNK, CHUNK), jnp.int32),
            pltpu.VMEM((NCHUNK, CHUNK), jnp.int32),
            pltpu.VMEM((NCHUNK, CHUNK, L), jnp.float32),
            pltpu.VMEM((8 * L,), jnp.float32),
            pltpu.VMEM((8 * L,), jnp.float32),
            pltpu.VMEM((NCHUNK, 4, 4, CHUNK), jnp.float32),
            pltpu.VMEM((NCHUNK, 3, 4, CHUNK), jnp.float32),
            pltpu.SemaphoreType.DMA,
        ],
        compiler_params=pltpu.CompilerParams(
            use_tc_tiling_on_sc=False, needs_layout_passes=False),
    )
    return f(rig, cam, proj, idx4)


def kernel(rig_t_world, camera_t_rig, projection, image_idx):
    nf = rig_t_world.shape[0]
    ncam = camera_t_rig.shape[0]
    b = image_idx.shape[0]

    rig = rig_t_world.reshape(nf, 16)
    cam = camera_t_rig.reshape(ncam * 16)
    projpad = jnp.concatenate(
        [projection.reshape(ncam, 9),
         jnp.zeros((ncam, 7), jnp.float32)], axis=1).reshape(ncam * 16)
    idx4 = image_idx.reshape(NW, NCHUNK, CHUNK)

    cwp, prjp = _sc_call(rig, cam, projpad, idx4)

    camera_t_world = cwp.transpose(1, 3, 0, 2).reshape(b, 4, 4)
    proj = prjp[:, :, :3, :].transpose(1, 3, 0, 2).reshape(b, 3, 3)
    return (camera_t_world, proj)
